# baseline (speedup 1.0000x reference)
import jax
import jax.numpy as jnp
from jax.experimental import pallas as pl
from jax.experimental.pallas import tpu as pltpu

_EMBED = 128
_Q = 65536
_B = 4096
_NCOPY = 8
_RPC = _EMBED // _NCOPY


def _dma_body(ptr_ref, norm_ref, q_any, out_any, normt, sem, slab_sem):
    copies = []
    for i in range(_NCOPY):
        rows = pl.ds(i * _RPC, _RPC)
        c = pltpu.make_async_copy(q_any.at[rows, :], out_any.at[rows, :], sem)
        c.start()
        copies.append(c)
    normt[...] = norm_ref[...].T
    for c in copies:
        c.wait()
    ptr = pl.multiple_of(ptr_ref[0], _B)
    slab = pltpu.make_async_copy(
        normt, out_any.at[:, pl.ds(ptr, _B)], slab_sem)
    slab.start()
    slab.wait()


def kernel(norm_vec, anorm_vec, temp, anorm_feats_queue, queue_ptr):
    new_queue = pl.pallas_call(
        _dma_body,
        grid_spec=pltpu.PrefetchScalarGridSpec(
            num_scalar_prefetch=1,
            grid=(1,),
            in_specs=[
                pl.BlockSpec((_B, _EMBED), lambda i, ptr: (0, 0)),
                pl.BlockSpec(memory_space=pltpu.HBM),
            ],
            out_specs=pl.BlockSpec(memory_space=pltpu.HBM),
            scratch_shapes=[
                pltpu.VMEM((_EMBED, _B), jnp.float32),
                pltpu.SemaphoreType.DMA,
                pltpu.SemaphoreType.DMA,
            ],
        ),
        out_shape=jax.ShapeDtypeStruct((_EMBED, _Q), jnp.float32),
    )(queue_ptr, norm_vec, anorm_feats_queue)
    new_ptr = ((queue_ptr + _B) % _Q).astype(jnp.int32)
    loss = jnp.asarray(0.0, dtype=jnp.float32)
    return loss, new_queue, new_ptr


# speedup vs baseline: 26.6940x; 26.6940x over previous
import jax
import jax.numpy as jnp
from jax.experimental import pallas as pl
from jax.experimental.pallas import tpu as pltpu

_EMBED = 128
_Q = 65536
_B = 4096
_C = 2048
_NB = _Q // _C
_SB = _B // _C


def _update_body(ptr_ref, norm_ref, q_ref, out_ref):
    i = pl.program_id(0)
    off = i - ptr_ref[0] // _C
    in_slab = jnp.logical_and(off >= 0, off < _SB)

    @pl.when(in_slab)
    def _():
        out_ref[...] = norm_ref[...].T

    @pl.when(jnp.logical_not(in_slab))
    def _():
        out_ref[...] = q_ref[...]


def _norm_index(i, ptr):
    off = i - ptr[0] // _C
    return (jnp.clip(off, 0, _SB - 1), 0)


def _q_index(i, ptr):
    off = i - ptr[0] // _C
    in_slab = jnp.logical_and(off >= 0, off < _SB)
    return (0, jnp.where(in_slab, jnp.maximum(i - _SB, 0), i))


def kernel(norm_vec, anorm_vec, temp, anorm_feats_queue, queue_ptr):
    grid_spec = pltpu.PrefetchScalarGridSpec(
        num_scalar_prefetch=1,
        grid=(_NB,),
        in_specs=[
            pl.BlockSpec((_C, _EMBED), _norm_index),
            pl.BlockSpec((_EMBED, _C), _q_index),
        ],
        out_specs=pl.BlockSpec((_EMBED, _C), lambda i, ptr: (0, i)),
    )
    new_queue = pl.pallas_call(
        _update_body,
        grid_spec=grid_spec,
        out_shape=jax.ShapeDtypeStruct((_EMBED, _Q), jnp.float32),
    )(queue_ptr, norm_vec, anorm_feats_queue)
    new_ptr = ((queue_ptr + _B) % _Q).astype(jnp.int32)
    loss = jnp.asarray(0.0, dtype=jnp.float32)
    return loss, new_queue, new_ptr


# speedup vs baseline: 37.6475x; 1.4103x over previous
import jax
import jax.numpy as jnp
from jax.experimental import pallas as pl
from jax.experimental.pallas import tpu as pltpu

_EMBED = 128
_Q = 65536
_B = 4096
_C = 8192
_NB = _Q // _C


def _update_body(ptr_ref, norm_ref, q_ref, out_ref):
    i = pl.program_id(0)
    ptr = ptr_ref[0]
    slab_blk = ptr // _C

    out_ref[...] = q_ref[...]

    @pl.when(i == slab_blk)
    def _():
        half = pl.multiple_of(ptr % _C, _B)
        out_ref[:, pl.ds(half, _B)] = norm_ref[...].T


def kernel(norm_vec, anorm_vec, temp, anorm_feats_queue, queue_ptr):
    grid_spec = pltpu.PrefetchScalarGridSpec(
        num_scalar_prefetch=1,
        grid=(_NB,),
        in_specs=[
            pl.BlockSpec((_B, _EMBED), lambda i, ptr: (0, 0)),
            pl.BlockSpec((_EMBED, _C), lambda i, ptr: (0, i)),
        ],
        out_specs=pl.BlockSpec((_EMBED, _C), lambda i, ptr: (0, i)),
    )
    new_queue = pl.pallas_call(
        _update_body,
        grid_spec=grid_spec,
        out_shape=jax.ShapeDtypeStruct((_EMBED, _Q), jnp.float32),
    )(queue_ptr, norm_vec, anorm_feats_queue)
    new_ptr = ((queue_ptr + _B) % _Q).astype(jnp.int32)
    loss = jnp.asarray(0.0, dtype=jnp.float32)
    return loss, new_queue, new_ptr


# speedup vs baseline: 40.2998x; 1.0705x over previous
import jax
import jax.numpy as jnp
from jax.experimental import pallas as pl
from jax.experimental.pallas import tpu as pltpu

_EMBED = 128
_Q = 65536
_B = 4096
_C = 16384
_NB = _Q // _C


def _update_body(ptr_ref, norm_ref, q_ref, out_ref):
    i = pl.program_id(0)
    ptr = ptr_ref[0]
    slab_blk = ptr // _C

    out_ref[...] = q_ref[...]

    @pl.when(i == slab_blk)
    def _():
        half = pl.multiple_of(ptr % _C, _B)
        out_ref[:, pl.ds(half, _B)] = norm_ref[...].T


def kernel(norm_vec, anorm_vec, temp, anorm_feats_queue, queue_ptr):
    grid_spec = pltpu.PrefetchScalarGridSpec(
        num_scalar_prefetch=1,
        grid=(_NB,),
        in_specs=[
            pl.BlockSpec((_B, _EMBED), lambda i, ptr: (0, 0)),
            pl.BlockSpec((_EMBED, _C), lambda i, ptr: (0, i)),
        ],
        out_specs=pl.BlockSpec((_EMBED, _C), lambda i, ptr: (0, i)),
    )
    new_queue = pl.pallas_call(
        _update_body,
        grid_spec=grid_spec,
        out_shape=jax.ShapeDtypeStruct((_EMBED, _Q), jnp.float32),
    )(queue_ptr, norm_vec, anorm_feats_queue)
    new_ptr = ((queue_ptr + _B) % _Q).astype(jnp.int32)
    loss = jnp.asarray(0.0, dtype=jnp.float32)
    return loss, new_queue, new_ptr
